# Initial kernel scaffold; baseline (speedup 1.0000x reference)
import jax
import jax.numpy as jnp
from jax.experimental import pallas as pl


def kernel(x, edge_index, edge_type, W1_rel, W1_root, b1, W2_rel, W2_root, b2, W_lin, b_lin):
    raise NotImplementedError("write your pallas kernel here")


# speedup vs baseline: 4.3181x; 4.3181x over previous
import functools

import jax
import jax.numpy as jnp
from jax import lax
from jax.experimental import pallas as pl
from jax.experimental.pallas import tpu as pltpu
from jax.experimental.pallas import tpu_sc as plsc

N_NODES = 10000
N_EDGES = 320000
D = 128
DF = 144
D_OUT_LL = 64

NC = 2
NS = 16
NW = NC * NS
EPW = N_EDGES // NW
CHUNK = 80
NCHUNKS = EPW // CHUNK
ROWS_PAD = 10112
ROWS_PER_TILE = ROWS_PAD // NS
DUMMY = N_NODES


def _sc_seg_kernel(rel, feat_h, src_h, dst_h, typ_h, zrows_h, agg_o,
                   t_v, i_v, j_v, idx_i, rows_v, sem, agg_sh):
  cid = lax.axis_index("c")
  sid = lax.axis_index("s")
  wid = cid * NS + sid
  ebase = wid * EPW
  rbase = sid * ROWS_PER_TILE

  pltpu.sync_copy(zrows_h, agg_sh.at[pl.ds(rbase, ROWS_PER_TILE)])
  plsc.subcore_barrier()

  def chunk_body(c, carry):
    base = ebase + c * CHUNK
    pltpu.sync_copy(typ_h.at[pl.ds(base, CHUNK)], t_v)
    pltpu.sync_copy(src_h.at[pl.ds(base, CHUNK)], i_v)
    pltpu.sync_copy(dst_h.at[pl.ds(base, CHUNK)], j_v)
    for t in range(CHUNK // 16):
      sl = pl.ds(t * 16, 16)
      m = t_v[sl] == rel
      idx_i[sl] = jnp.where(m, i_v[sl], DUMMY)
    pltpu.async_copy(feat_h.at[j_v], rows_v, sem).wait()
    pltpu.sync_copy(rows_v, agg_sh.at[idx_i], add=True)
    return carry

  lax.fori_loop(0, NCHUNKS, chunk_body, 0)

  plsc.subcore_barrier()
  pltpu.sync_copy(agg_sh.at[pl.ds(rbase, ROWS_PER_TILE)],
                  agg_o.at[cid, pl.ds(rbase, ROWS_PER_TILE)])


def _make_sc_call(rel):
  mesh = plsc.VectorSubcoreMesh(core_axis_name="c", subcore_axis_name="s",
                                num_cores=NC, num_subcores=NS)
  return pl.kernel(
      functools.partial(_sc_seg_kernel, rel),
      out_type=jax.ShapeDtypeStruct((NC, ROWS_PAD, DF), jnp.float32),
      mesh=mesh,
      scratch_types=[
          pltpu.VMEM((CHUNK,), jnp.int32),
          pltpu.VMEM((CHUNK,), jnp.int32),
          pltpu.VMEM((CHUNK,), jnp.int32),
          pltpu.VMEM((CHUNK,), jnp.int32),
          pltpu.VMEM((CHUNK, DF), jnp.float32),
          pltpu.SemaphoreType.DMA,
          pltpu.VMEM_SHARED((ROWS_PAD, DF), jnp.float32),
      ],
      name=f"rgcn_seg_rel{rel}",
      compiler_params=pltpu.CompilerParams(needs_layout_passes=False, use_tc_tiling_on_sc=False),
  )


_BLK = 2000


def _tc_layer_body(aggA, aggB, feat, wr, wo, b, out):
  agg = aggA[...] + aggB[...]
  deg = agg[:, D:D + 1]
  norm = jnp.where(deg > 0, 1.0 / jnp.maximum(deg, 1.0), 0.0)
  h = (jnp.dot(agg[:, :D] * norm, wr[...], preferred_element_type=jnp.float32)
       + jnp.dot(feat[:, :D], wo[...], preferred_element_type=jnp.float32)
       + b[...])
  h = jnp.maximum(h, 0.0)
  ones = jnp.ones((h.shape[0], 1), jnp.float32)
  pad = jnp.zeros((h.shape[0], DF - D - 1), jnp.float32)
  out[...] = jnp.concatenate([h, ones, pad], axis=1)


def _tc_layer(aggA, aggB, feat, wr, wo, b):
  grid = (N_NODES // _BLK,)
  return pl.pallas_call(
      _tc_layer_body,
      grid=grid,
      in_specs=[
          pl.BlockSpec((_BLK, DF), lambda i: (i, 0)),
          pl.BlockSpec((_BLK, DF), lambda i: (i, 0)),
          pl.BlockSpec((_BLK, DF), lambda i: (i, 0)),
          pl.BlockSpec((D, D), lambda i: (0, 0)),
          pl.BlockSpec((D, D), lambda i: (0, 0)),
          pl.BlockSpec((1, D), lambda i: (0, 0)),
      ],
      out_specs=pl.BlockSpec((_BLK, DF), lambda i: (i, 0)),
      out_shape=jax.ShapeDtypeStruct((N_NODES, DF), jnp.float32),
  )(aggA, aggB, feat, wr, wo, b)


def _tc_final_body(aggA, aggB, feat, wr, wo, b, wl, bl, out):
  agg = aggA[...] + aggB[...]
  deg = agg[:, D:D + 1]
  norm = jnp.where(deg > 0, 1.0 / jnp.maximum(deg, 1.0), 0.0)
  h = (jnp.dot(agg[:, :D] * norm, wr[...], preferred_element_type=jnp.float32)
       + jnp.dot(feat[:, :D], wo[...], preferred_element_type=jnp.float32)
       + b[...])
  h = jnp.maximum(h, 0.0)
  y = jnp.dot(h, wl[...], preferred_element_type=jnp.float32) + bl[...]
  mx = jnp.max(y, axis=1, keepdims=True)
  lse = jnp.log(jnp.sum(jnp.exp(y - mx), axis=1, keepdims=True))
  out[...] = y - mx - lse


def _tc_final(aggA, aggB, feat, wr, wo, b, wl, bl):
  grid = (N_NODES // _BLK,)
  return pl.pallas_call(
      _tc_final_body,
      grid=grid,
      in_specs=[
          pl.BlockSpec((_BLK, DF), lambda i: (i, 0)),
          pl.BlockSpec((_BLK, DF), lambda i: (i, 0)),
          pl.BlockSpec((_BLK, DF), lambda i: (i, 0)),
          pl.BlockSpec((D, D), lambda i: (0, 0)),
          pl.BlockSpec((D, D), lambda i: (0, 0)),
          pl.BlockSpec((1, D), lambda i: (0, 0)),
          pl.BlockSpec((D, D_OUT_LL), lambda i: (0, 0)),
          pl.BlockSpec((1, D_OUT_LL), lambda i: (0, 0)),
      ],
      out_specs=pl.BlockSpec((_BLK, D_OUT_LL), lambda i: (i, 0)),
      out_shape=jax.ShapeDtypeStruct((N_NODES, D_OUT_LL), jnp.float32),
  )(aggA, aggB, feat, wr, wo, b, wl, bl)


def kernel(x, edge_index, edge_type, W1_rel, W1_root, b1, W2_rel, W2_root, b2,
           W_lin, b_lin):
  src = edge_index[0].astype(jnp.int32)
  dst = edge_index[1].astype(jnp.int32)
  typ = edge_type.astype(jnp.int32)

  ones_col = jnp.ones((N_NODES, 1), jnp.float32)
  pad_cols = jnp.zeros((N_NODES, DF - D - 1), jnp.float32)
  x_pad = jnp.concatenate([x, ones_col, pad_cols], axis=1)

  zrows = jnp.zeros((ROWS_PER_TILE, DF), jnp.float32)

  sc0 = _make_sc_call(0)
  sc1 = _make_sc_call(1)

  agg = sc0(x_pad, src, dst, typ, zrows)
  h1 = _tc_layer(agg[0, :N_NODES], agg[1, :N_NODES],
                 x_pad, W1_rel[0], W1_root, b1[None, :])

  agg2 = sc1(h1, src, dst, typ, zrows)
  out = _tc_final(agg2[0, :N_NODES], agg2[1, :N_NODES],
                  h1, W2_rel[1], W2_root, b2[None, :],
                  W_lin, b_lin[None, :])
  return out


# speedup vs baseline: 7.4484x; 1.7249x over previous
import functools

import jax
import jax.numpy as jnp
from jax import lax
from jax.experimental import pallas as pl
from jax.experimental.pallas import tpu as pltpu
from jax.experimental.pallas import tpu_sc as plsc

N_NODES = 10000
N_EDGES = 320000
D = 128
DF = 144
D_OUT_LL = 64

NC = 2
NS = 16
NW = NC * NS
EPW = N_EDGES // NW
SEG = 2000
NSEG = EPW // SEG
CHUNK = 80
CPS = SEG // CHUNK
ROWS_PAD = 10112
ROWS_PER_TILE = ROWS_PAD // NS
DUMMY = N_NODES


def _sc_seg_kernel(rel, feat_h, src_h, dst_h, typ_h, zrows_h, agg_o,
                   t0, t1, i0, i1, j0, j1, x0, x1, r0, r1,
                   sS0, sS1, sG0, sG1, sW0, sW1, agg_sh):
  cid = lax.axis_index("c")
  sid = lax.axis_index("s")
  wid = cid * NS + sid
  ebase = wid * EPW
  rbase = sid * ROWS_PER_TILE

  tb = (t0, t1)
  ib = (i0, i1)
  jb = (j0, j1)
  xb = (x0, x1)
  rb = (r0, r1)
  sS = (sS0, sS1)
  sG = (sG0, sG1)
  sW = (sW0, sW1)

  pltpu.sync_copy(zrows_h, agg_sh.at[pl.ds(rbase, ROWS_PER_TILE)])
  plsc.subcore_barrier()

  def stage(s):
    sp = s % 2
    base = ebase + s * SEG
    return [pltpu.async_copy(typ_h.at[pl.ds(base, SEG)], tb[sp], sS[sp]),
            pltpu.async_copy(src_h.at[pl.ds(base, SEG)], ib[sp], sS[sp]),
            pltpu.async_copy(dst_h.at[pl.ds(base, SEG)], jb[sp], sS[sp])]

  pend_stage = stage(0)
  pend_gather = None
  pend_scatter = [None, None]
  g = 0
  for s in range(NSEG):
    sp = s % 2
    for d in pend_stage:
      d.wait()
    if pend_gather is not None:
      q, qd = pend_gather
      qd.wait()
      pend_scatter[q] = pltpu.async_copy(
          rb[q], agg_sh.at[xb[q]], sW[q], add=True)
      pend_gather = None
    if s + 1 < NSEG:
      pend_stage = stage(s + 1)
    for c in range(CPS):
      p = g % 2
      if pend_scatter[p] is not None:
        pend_scatter[p].wait()
      for t in range(CHUNK // 16):
        sl = pl.ds(c * CHUNK + t * 16, 16)
        m = tb[sp][sl] == rel
        xb[p][pl.ds(t * 16, 16)] = jnp.where(m, ib[sp][sl], DUMMY)
      gd = pltpu.async_copy(
          feat_h.at[jb[sp].at[pl.ds(c * CHUNK, CHUNK)]], rb[p], sG[p])
      if pend_gather is not None:
        q, qd = pend_gather
        qd.wait()
        pend_scatter[q] = pltpu.async_copy(
            rb[q], agg_sh.at[xb[q]], sW[q], add=True)
      pend_gather = (p, gd)
      g += 1
  q, qd = pend_gather
  qd.wait()
  pend_scatter[q] = pltpu.async_copy(rb[q], agg_sh.at[xb[q]], sW[q], add=True)
  for p in range(2):
    if pend_scatter[p] is not None:
      pend_scatter[p].wait()

  plsc.subcore_barrier()
  pltpu.sync_copy(agg_sh.at[pl.ds(rbase, ROWS_PER_TILE)],
                  agg_o.at[cid, pl.ds(rbase, ROWS_PER_TILE)])


def _make_sc_call(rel):
  mesh = plsc.VectorSubcoreMesh(core_axis_name="c", subcore_axis_name="s",
                                num_cores=NC, num_subcores=NS)
  return pl.kernel(
      functools.partial(_sc_seg_kernel, rel),
      out_type=jax.ShapeDtypeStruct((NC, ROWS_PAD, DF), jnp.float32),
      mesh=mesh,
      scratch_types=[
          pltpu.VMEM((SEG,), jnp.int32),
          pltpu.VMEM((SEG,), jnp.int32),
          pltpu.VMEM((SEG,), jnp.int32),
          pltpu.VMEM((SEG,), jnp.int32),
          pltpu.VMEM((SEG,), jnp.int32),
          pltpu.VMEM((SEG,), jnp.int32),
          pltpu.VMEM((CHUNK,), jnp.int32),
          pltpu.VMEM((CHUNK,), jnp.int32),
          pltpu.VMEM((CHUNK, DF), jnp.float32),
          pltpu.VMEM((CHUNK, DF), jnp.float32),
          pltpu.SemaphoreType.DMA,
          pltpu.SemaphoreType.DMA,
          pltpu.SemaphoreType.DMA,
          pltpu.SemaphoreType.DMA,
          pltpu.SemaphoreType.DMA,
          pltpu.SemaphoreType.DMA,
          pltpu.VMEM_SHARED((ROWS_PAD, DF), jnp.float32),
      ],
      name=f"rgcn_seg_rel{rel}",
      compiler_params=pltpu.CompilerParams(needs_layout_passes=False, use_tc_tiling_on_sc=False),
  )


_BLK = 2000


def _tc_layer_body(aggA, aggB, feat, wr, wo, b, out):
  agg = aggA[...] + aggB[...]
  deg = agg[:, D:D + 1]
  norm = jnp.where(deg > 0, 1.0 / jnp.maximum(deg, 1.0), 0.0)
  h = (jnp.dot(agg[:, :D] * norm, wr[...], preferred_element_type=jnp.float32)
       + jnp.dot(feat[:, :D], wo[...], preferred_element_type=jnp.float32)
       + b[...])
  h = jnp.maximum(h, 0.0)
  ones = jnp.ones((h.shape[0], 1), jnp.float32)
  pad = jnp.zeros((h.shape[0], DF - D - 1), jnp.float32)
  out[...] = jnp.concatenate([h, ones, pad], axis=1)


def _tc_layer(aggA, aggB, feat, wr, wo, b):
  grid = (N_NODES // _BLK,)
  return pl.pallas_call(
      _tc_layer_body,
      grid=grid,
      in_specs=[
          pl.BlockSpec((_BLK, DF), lambda i: (i, 0)),
          pl.BlockSpec((_BLK, DF), lambda i: (i, 0)),
          pl.BlockSpec((_BLK, DF), lambda i: (i, 0)),
          pl.BlockSpec((D, D), lambda i: (0, 0)),
          pl.BlockSpec((D, D), lambda i: (0, 0)),
          pl.BlockSpec((1, D), lambda i: (0, 0)),
      ],
      out_specs=pl.BlockSpec((_BLK, DF), lambda i: (i, 0)),
      out_shape=jax.ShapeDtypeStruct((N_NODES, DF), jnp.float32),
  )(aggA, aggB, feat, wr, wo, b)


def _tc_final_body(aggA, aggB, feat, wr, wo, b, wl, bl, out):
  agg = aggA[...] + aggB[...]
  deg = agg[:, D:D + 1]
  norm = jnp.where(deg > 0, 1.0 / jnp.maximum(deg, 1.0), 0.0)
  h = (jnp.dot(agg[:, :D] * norm, wr[...], preferred_element_type=jnp.float32)
       + jnp.dot(feat[:, :D], wo[...], preferred_element_type=jnp.float32)
       + b[...])
  h = jnp.maximum(h, 0.0)
  y = jnp.dot(h, wl[...], preferred_element_type=jnp.float32) + bl[...]
  mx = jnp.max(y, axis=1, keepdims=True)
  lse = jnp.log(jnp.sum(jnp.exp(y - mx), axis=1, keepdims=True))
  out[...] = y - mx - lse


def _tc_final(aggA, aggB, feat, wr, wo, b, wl, bl):
  grid = (N_NODES // _BLK,)
  return pl.pallas_call(
      _tc_final_body,
      grid=grid,
      in_specs=[
          pl.BlockSpec((_BLK, DF), lambda i: (i, 0)),
          pl.BlockSpec((_BLK, DF), lambda i: (i, 0)),
          pl.BlockSpec((_BLK, DF), lambda i: (i, 0)),
          pl.BlockSpec((D, D), lambda i: (0, 0)),
          pl.BlockSpec((D, D), lambda i: (0, 0)),
          pl.BlockSpec((1, D), lambda i: (0, 0)),
          pl.BlockSpec((D, D_OUT_LL), lambda i: (0, 0)),
          pl.BlockSpec((1, D_OUT_LL), lambda i: (0, 0)),
      ],
      out_specs=pl.BlockSpec((_BLK, D_OUT_LL), lambda i: (i, 0)),
      out_shape=jax.ShapeDtypeStruct((N_NODES, D_OUT_LL), jnp.float32),
  )(aggA, aggB, feat, wr, wo, b, wl, bl)


def kernel(x, edge_index, edge_type, W1_rel, W1_root, b1, W2_rel, W2_root, b2,
           W_lin, b_lin):
  src = edge_index[0].astype(jnp.int32)
  dst = edge_index[1].astype(jnp.int32)
  typ = edge_type.astype(jnp.int32)

  ones_col = jnp.ones((N_NODES, 1), jnp.float32)
  pad_cols = jnp.zeros((N_NODES, DF - D - 1), jnp.float32)
  x_pad = jnp.concatenate([x, ones_col, pad_cols], axis=1)

  zrows = jnp.zeros((ROWS_PER_TILE, DF), jnp.float32)

  sc0 = _make_sc_call(0)
  sc1 = _make_sc_call(1)

  agg = sc0(x_pad, src, dst, typ, zrows)
  h1 = _tc_layer(agg[0, :N_NODES], agg[1, :N_NODES],
                 x_pad, W1_rel[0], W1_root, b1[None, :])

  agg2 = sc1(h1, src, dst, typ, zrows)
  out = _tc_final(agg2[0, :N_NODES], agg2[1, :N_NODES],
                  h1, W2_rel[1], W2_root, b2[None, :],
                  W_lin, b_lin[None, :])
  return out


# speedup vs baseline: 9.2536x; 1.2424x over previous
import functools

import jax
import jax.numpy as jnp
from jax import lax
from jax.experimental import pallas as pl
from jax.experimental.pallas import tpu as pltpu
from jax.experimental.pallas import tpu_sc as plsc

N_NODES = 10000
N_EDGES = 320000
D = 128
DF = 144
D_OUT_LL = 64

NC = 2
NS = 16
NW = NC * NS
EPW = N_EDGES // NW
SEG = 2000
NSEG = EPW // SEG
CHUNK = 80
CPS = SEG // CHUNK
ROWS_PAD = 10112
ROWS_PER_TILE = ROWS_PAD // NS
DUMMY = N_NODES


def _sc_seg_kernel(rel, feat_h, src_h, dst_h, typ_h, zrows_h, agg_o,
                   t0, t1, i0, i1, j0, j1, x0, x1, r0, r1,
                   sS0, sS1, sG0, sG1, sW0, sW1, agg_sh):
  cid = lax.axis_index("c")
  sid = lax.axis_index("s")
  wid = cid * NS + sid
  ebase = wid * EPW
  rbase = sid * ROWS_PER_TILE

  tb = (t0, t1)
  ib = (i0, i1)
  jb = (j0, j1)
  xb = (x0, x1)
  rb = (r0, r1)
  sS = (sS0, sS1)
  sG = (sG0, sG1)
  sW = (sW0, sW1)

  pltpu.sync_copy(zrows_h, agg_sh.at[pl.ds(rbase, ROWS_PER_TILE)])
  plsc.subcore_barrier()

  lane = lax.iota(jnp.int32, 16)

  def stage(s):
    sp = s % 2
    base = ebase + s * SEG
    return [pltpu.async_copy(typ_h.at[pl.ds(base, SEG)], tb[sp], sS[sp]),
            pltpu.async_copy(src_h.at[pl.ds(base, SEG)], ib[sp], sS[sp]),
            pltpu.async_copy(dst_h.at[pl.ds(base, SEG)], jb[sp], sS[sp])]

  pend_stage = stage(0)
  pend_gather = None
  pend_scatter = [None, None]
  g = 0
  for s in range(NSEG):
    sp = s % 2
    for d in pend_stage:
      d.wait()
    if pend_gather is not None:
      q, qd = pend_gather
      qd.wait()
      pend_scatter[q] = pltpu.async_copy(
          rb[q], agg_sh.at[xb[q]], sW[q], add=True)
      pend_gather = None
    if s + 1 < NSEG:
      pend_stage = stage(s + 1)
    for c in range(CPS):
      p = g % 2
      if pend_scatter[p] is not None:
        pend_scatter[p].wait()
      for t in range(CHUNK // 16):
        sl = pl.ds(c * CHUNK + t * 16, 16)
        m = tb[sp][sl] == rel
        dummy_v = lane + (DUMMY + t * 16)
        xb[p][pl.ds(t * 16, 16)] = jnp.where(m, ib[sp][sl], dummy_v)
      gd = pltpu.async_copy(
          feat_h.at[jb[sp].at[pl.ds(c * CHUNK, CHUNK)]], rb[p], sG[p])
      if pend_gather is not None:
        q, qd = pend_gather
        qd.wait()
        pend_scatter[q] = pltpu.async_copy(
            rb[q], agg_sh.at[xb[q]], sW[q], add=True)
      pend_gather = (p, gd)
      g += 1
  q, qd = pend_gather
  qd.wait()
  pend_scatter[q] = pltpu.async_copy(rb[q], agg_sh.at[xb[q]], sW[q], add=True)
  for p in range(2):
    if pend_scatter[p] is not None:
      pend_scatter[p].wait()

  plsc.subcore_barrier()
  pltpu.sync_copy(agg_sh.at[pl.ds(rbase, ROWS_PER_TILE)],
                  agg_o.at[cid, pl.ds(rbase, ROWS_PER_TILE)])


def _make_sc_call(rel):
  mesh = plsc.VectorSubcoreMesh(core_axis_name="c", subcore_axis_name="s",
                                num_cores=NC, num_subcores=NS)
  return pl.kernel(
      functools.partial(_sc_seg_kernel, rel),
      out_type=jax.ShapeDtypeStruct((NC, ROWS_PAD, DF), jnp.float32),
      mesh=mesh,
      scratch_types=[
          pltpu.VMEM((SEG,), jnp.int32),
          pltpu.VMEM((SEG,), jnp.int32),
          pltpu.VMEM((SEG,), jnp.int32),
          pltpu.VMEM((SEG,), jnp.int32),
          pltpu.VMEM((SEG,), jnp.int32),
          pltpu.VMEM((SEG,), jnp.int32),
          pltpu.VMEM((CHUNK,), jnp.int32),
          pltpu.VMEM((CHUNK,), jnp.int32),
          pltpu.VMEM((CHUNK, DF), jnp.float32),
          pltpu.VMEM((CHUNK, DF), jnp.float32),
          pltpu.SemaphoreType.DMA,
          pltpu.SemaphoreType.DMA,
          pltpu.SemaphoreType.DMA,
          pltpu.SemaphoreType.DMA,
          pltpu.SemaphoreType.DMA,
          pltpu.SemaphoreType.DMA,
          pltpu.VMEM_SHARED((ROWS_PAD, DF), jnp.float32),
      ],
      name=f"rgcn_seg_rel{rel}",
      compiler_params=pltpu.CompilerParams(needs_layout_passes=False, use_tc_tiling_on_sc=False),
  )


_BLK = 2000


def _tc_layer_body(aggA, aggB, feat, wr, wo, b, out):
  agg = aggA[...] + aggB[...]
  deg = agg[:, D:D + 1]
  norm = jnp.where(deg > 0, 1.0 / jnp.maximum(deg, 1.0), 0.0)
  h = (jnp.dot(agg[:, :D] * norm, wr[...], preferred_element_type=jnp.float32)
       + jnp.dot(feat[:, :D], wo[...], preferred_element_type=jnp.float32)
       + b[...])
  h = jnp.maximum(h, 0.0)
  ones = jnp.ones((h.shape[0], 1), jnp.float32)
  pad = jnp.zeros((h.shape[0], DF - D - 1), jnp.float32)
  out[...] = jnp.concatenate([h, ones, pad], axis=1)


def _tc_layer(aggA, aggB, feat, wr, wo, b):
  grid = (N_NODES // _BLK,)
  return pl.pallas_call(
      _tc_layer_body,
      grid=grid,
      in_specs=[
          pl.BlockSpec((_BLK, DF), lambda i: (i, 0)),
          pl.BlockSpec((_BLK, DF), lambda i: (i, 0)),
          pl.BlockSpec((_BLK, DF), lambda i: (i, 0)),
          pl.BlockSpec((D, D), lambda i: (0, 0)),
          pl.BlockSpec((D, D), lambda i: (0, 0)),
          pl.BlockSpec((1, D), lambda i: (0, 0)),
      ],
      out_specs=pl.BlockSpec((_BLK, DF), lambda i: (i, 0)),
      out_shape=jax.ShapeDtypeStruct((N_NODES, DF), jnp.float32),
  )(aggA, aggB, feat, wr, wo, b)


def _tc_final_body(aggA, aggB, feat, wr, wo, b, wl, bl, out):
  agg = aggA[...] + aggB[...]
  deg = agg[:, D:D + 1]
  norm = jnp.where(deg > 0, 1.0 / jnp.maximum(deg, 1.0), 0.0)
  h = (jnp.dot(agg[:, :D] * norm, wr[...], preferred_element_type=jnp.float32)
       + jnp.dot(feat[:, :D], wo[...], preferred_element_type=jnp.float32)
       + b[...])
  h = jnp.maximum(h, 0.0)
  y = jnp.dot(h, wl[...], preferred_element_type=jnp.float32) + bl[...]
  mx = jnp.max(y, axis=1, keepdims=True)
  lse = jnp.log(jnp.sum(jnp.exp(y - mx), axis=1, keepdims=True))
  out[...] = y - mx - lse


def _tc_final(aggA, aggB, feat, wr, wo, b, wl, bl):
  grid = (N_NODES // _BLK,)
  return pl.pallas_call(
      _tc_final_body,
      grid=grid,
      in_specs=[
          pl.BlockSpec((_BLK, DF), lambda i: (i, 0)),
          pl.BlockSpec((_BLK, DF), lambda i: (i, 0)),
          pl.BlockSpec((_BLK, DF), lambda i: (i, 0)),
          pl.BlockSpec((D, D), lambda i: (0, 0)),
          pl.BlockSpec((D, D), lambda i: (0, 0)),
          pl.BlockSpec((1, D), lambda i: (0, 0)),
          pl.BlockSpec((D, D_OUT_LL), lambda i: (0, 0)),
          pl.BlockSpec((1, D_OUT_LL), lambda i: (0, 0)),
      ],
      out_specs=pl.BlockSpec((_BLK, D_OUT_LL), lambda i: (i, 0)),
      out_shape=jax.ShapeDtypeStruct((N_NODES, D_OUT_LL), jnp.float32),
  )(aggA, aggB, feat, wr, wo, b, wl, bl)


def kernel(x, edge_index, edge_type, W1_rel, W1_root, b1, W2_rel, W2_root, b2,
           W_lin, b_lin):
  src = edge_index[0].astype(jnp.int32)
  dst = edge_index[1].astype(jnp.int32)
  typ = edge_type.astype(jnp.int32)

  ones_col = jnp.ones((N_NODES, 1), jnp.float32)
  pad_cols = jnp.zeros((N_NODES, DF - D - 1), jnp.float32)
  x_pad = jnp.concatenate([x, ones_col, pad_cols], axis=1)

  zrows = jnp.zeros((ROWS_PER_TILE, DF), jnp.float32)

  sc0 = _make_sc_call(0)
  sc1 = _make_sc_call(1)

  agg = sc0(x_pad, src, dst, typ, zrows)
  h1 = _tc_layer(agg[0, :N_NODES], agg[1, :N_NODES],
                 x_pad, W1_rel[0], W1_root, b1[None, :])

  agg2 = sc1(h1, src, dst, typ, zrows)
  out = _tc_final(agg2[0, :N_NODES], agg2[1, :N_NODES],
                  h1, W2_rel[1], W2_root, b2[None, :],
                  W_lin, b_lin[None, :])
  return out


# speedup vs baseline: 9.8866x; 1.0684x over previous
import functools

import jax
import jax.numpy as jnp
from jax import lax
from jax.experimental import pallas as pl
from jax.experimental.pallas import tpu as pltpu
from jax.experimental.pallas import tpu_sc as plsc

N_NODES = 10000
N_EDGES = 320000
D = 128
DF = 144
D_OUT_LL = 64

NC = 2
NS = 16
NW = NC * NS
EPW = N_EDGES // NW
SEG = 2000
NSEG = EPW // SEG
CHUNK = 80
CPS = SEG // CHUNK
ROWS_PAD = 10112
ROWS_PER_TILE = ROWS_PAD // NS
DUMMY = N_NODES


def _sc_seg_kernel(rel, feat_h, src_h, dst_h, typ_h, zrows_h, agg_o,
                   t0, t1, i0, i1, j0, j1, x0, x1, r0, r1,
                   sS0, sS1, sG0, sG1, sW0, sW1, agg_sh):
  cid = lax.axis_index("c")
  sid = lax.axis_index("s")
  wid = cid * NS + sid
  ebase = wid * EPW
  rbase = sid * ROWS_PER_TILE

  tb = (t0, t1)
  ib = (i0, i1)
  jb = (j0, j1)
  xb = (x0, x1)
  rb = (r0, r1)
  sS = (sS0, sS1)
  sG = (sG0, sG1)
  sW = (sW0, sW1)

  pltpu.sync_copy(zrows_h, agg_sh.at[pl.ds(rbase, ROWS_PER_TILE)])
  plsc.subcore_barrier()

  lane = lax.iota(jnp.int32, 16)

  def stage(s):
    sp = s % 2
    base = ebase + s * SEG
    return [pltpu.async_copy(typ_h.at[pl.ds(base, SEG)], tb[sp], sS[sp]),
            pltpu.async_copy(src_h.at[pl.ds(base, SEG)], ib[sp], sS[sp]),
            pltpu.async_copy(dst_h.at[pl.ds(base, SEG)], jb[sp], sS[sp])]

  pend_stage = stage(0)
  pend_gather = None
  pend_scatter = [None, None]
  g = 0
  for s in range(NSEG):
    sp = s % 2
    for d in pend_stage:
      d.wait()
    if pend_gather is not None:
      q, qd = pend_gather
      qd.wait()
      pend_scatter[q] = pltpu.async_copy(
          rb[q], agg_sh.at[xb[q]], sW[q], add=True)
      pend_gather = None
    if s + 1 < NSEG:
      pend_stage = stage(s + 1)
    for c in range(CPS):
      p = g % 2
      if pend_scatter[p] is not None:
        pend_scatter[p].wait()
      for t in range(CHUNK // 16):
        sl = pl.ds(c * CHUNK + t * 16, 16)
        m = tb[sp][sl] == rel
        dummy_v = lane + (DUMMY + t * 16)
        xb[p][pl.ds(t * 16, 16)] = jnp.where(m, ib[sp][sl], dummy_v)
      gd = pltpu.async_copy(
          feat_h.at[jb[sp].at[pl.ds(c * CHUNK, CHUNK)]], rb[p], sG[p])
      if pend_gather is not None:
        q, qd = pend_gather
        qd.wait()
        pend_scatter[q] = pltpu.async_copy(
            rb[q], agg_sh.at[xb[q]], sW[q], add=True)
      pend_gather = (p, gd)
      g += 1
  q, qd = pend_gather
  qd.wait()
  pend_scatter[q] = pltpu.async_copy(rb[q], agg_sh.at[xb[q]], sW[q], add=True)
  for p in range(2):
    if pend_scatter[p] is not None:
      pend_scatter[p].wait()

  plsc.subcore_barrier()
  pltpu.sync_copy(agg_sh.at[pl.ds(rbase, ROWS_PER_TILE)],
                  agg_o.at[cid, pl.ds(rbase, ROWS_PER_TILE)])


def _make_sc_call(rel):
  mesh = plsc.VectorSubcoreMesh(core_axis_name="c", subcore_axis_name="s",
                                num_cores=NC, num_subcores=NS)
  return pl.kernel(
      functools.partial(_sc_seg_kernel, rel),
      out_type=jax.ShapeDtypeStruct((NC, ROWS_PAD, DF), jnp.float32),
      mesh=mesh,
      scratch_types=[
          pltpu.VMEM((SEG,), jnp.int32),
          pltpu.VMEM((SEG,), jnp.int32),
          pltpu.VMEM((SEG,), jnp.int32),
          pltpu.VMEM((SEG,), jnp.int32),
          pltpu.VMEM((SEG,), jnp.int32),
          pltpu.VMEM((SEG,), jnp.int32),
          pltpu.VMEM((CHUNK,), jnp.int32),
          pltpu.VMEM((CHUNK,), jnp.int32),
          pltpu.VMEM((CHUNK, DF), jnp.float32),
          pltpu.VMEM((CHUNK, DF), jnp.float32),
          pltpu.SemaphoreType.DMA,
          pltpu.SemaphoreType.DMA,
          pltpu.SemaphoreType.DMA,
          pltpu.SemaphoreType.DMA,
          pltpu.SemaphoreType.DMA,
          pltpu.SemaphoreType.DMA,
          pltpu.VMEM_SHARED((ROWS_PAD, DF), jnp.float32),
      ],
      name=f"rgcn_seg_rel{rel}",
      compiler_params=pltpu.CompilerParams(needs_layout_passes=False, use_tc_tiling_on_sc=False),
  )


_BLK = 2000
_BLKP = 2528


def _tc_layer_body(aggP, feat, wr, wo, b, out):
  agg = aggP[0] + aggP[1]
  deg = agg[:, D:D + 1]
  norm = jnp.where(deg > 0, 1.0 / jnp.maximum(deg, 1.0), 0.0)
  h = (jnp.dot(agg[:, :D] * norm, wr[...], preferred_element_type=jnp.float32)
       + jnp.dot(feat[:, :D], wo[...], preferred_element_type=jnp.float32)
       + b[...])
  h = jnp.maximum(h, 0.0)
  ones = jnp.ones((h.shape[0], 1), jnp.float32)
  pad = jnp.zeros((h.shape[0], DF - D - 1), jnp.float32)
  out[...] = jnp.concatenate([h, ones, pad], axis=1)


def _tc_layer(agg, feat, wr, wo, b):
  grid = (ROWS_PAD // _BLKP,)
  return pl.pallas_call(
      _tc_layer_body,
      grid=grid,
      in_specs=[
          pl.BlockSpec((NC, _BLKP, DF), lambda i: (0, i, 0)),
          pl.BlockSpec((_BLKP, DF), lambda i: (i, 0)),
          pl.BlockSpec((D, D), lambda i: (0, 0)),
          pl.BlockSpec((D, D), lambda i: (0, 0)),
          pl.BlockSpec((1, D), lambda i: (0, 0)),
      ],
      out_specs=pl.BlockSpec((_BLKP, DF), lambda i: (i, 0)),
      out_shape=jax.ShapeDtypeStruct((ROWS_PAD, DF), jnp.float32),
  )(agg, feat, wr, wo, b)


def _tc_final_body(aggP, feat, wr, wo, b, wl, bl, out):
  agg = aggP[0] + aggP[1]
  deg = agg[:, D:D + 1]
  norm = jnp.where(deg > 0, 1.0 / jnp.maximum(deg, 1.0), 0.0)
  h = (jnp.dot(agg[:, :D] * norm, wr[...], preferred_element_type=jnp.float32)
       + jnp.dot(feat[:, :D], wo[...], preferred_element_type=jnp.float32)
       + b[...])
  h = jnp.maximum(h, 0.0)
  y = jnp.dot(h, wl[...], preferred_element_type=jnp.float32) + bl[...]
  mx = jnp.max(y, axis=1, keepdims=True)
  lse = jnp.log(jnp.sum(jnp.exp(y - mx), axis=1, keepdims=True))
  out[...] = y - mx - lse


def _tc_final(agg, feat, wr, wo, b, wl, bl):
  grid = (N_NODES // _BLK,)
  return pl.pallas_call(
      _tc_final_body,
      grid=grid,
      in_specs=[
          pl.BlockSpec((NC, _BLK, DF), lambda i: (0, i, 0)),
          pl.BlockSpec((_BLK, DF), lambda i: (i, 0)),
          pl.BlockSpec((D, D), lambda i: (0, 0)),
          pl.BlockSpec((D, D), lambda i: (0, 0)),
          pl.BlockSpec((1, D), lambda i: (0, 0)),
          pl.BlockSpec((D, D_OUT_LL), lambda i: (0, 0)),
          pl.BlockSpec((1, D_OUT_LL), lambda i: (0, 0)),
      ],
      out_specs=pl.BlockSpec((_BLK, D_OUT_LL), lambda i: (i, 0)),
      out_shape=jax.ShapeDtypeStruct((N_NODES, D_OUT_LL), jnp.float32),
  )(agg, feat, wr, wo, b, wl, bl)


def kernel(x, edge_index, edge_type, W1_rel, W1_root, b1, W2_rel, W2_root, b2,
           W_lin, b_lin):
  src = edge_index[0].astype(jnp.int32)
  dst = edge_index[1].astype(jnp.int32)
  typ = edge_type.astype(jnp.int32)

  ones_col = jnp.ones((N_NODES, 1), jnp.float32)
  pad_cols = jnp.zeros((N_NODES, DF - D - 1), jnp.float32)
  x_pad = jnp.concatenate([x, ones_col, pad_cols], axis=1)

  zrows = jnp.zeros((ROWS_PER_TILE, DF), jnp.float32)

  sc0 = _make_sc_call(0)
  sc1 = _make_sc_call(1)

  x_padf = jnp.concatenate(
      [x_pad, jnp.zeros((ROWS_PAD - N_NODES, DF), jnp.float32)], axis=0)

  agg = sc0(x_padf, src, dst, typ, zrows)
  h1 = _tc_layer(agg, x_padf, W1_rel[0], W1_root, b1[None, :])

  agg2 = sc1(h1, src, dst, typ, zrows)
  out = _tc_final(agg2, h1, W2_rel[1], W2_root, b2[None, :],
                  W_lin, b_lin[None, :])
  return out
